# Initial kernel scaffold; baseline (speedup 1.0000x reference)
import jax
import jax.numpy as jnp
from jax.experimental import pallas as pl


def kernel(hidden, gate_logits, W, b):
    raise NotImplementedError("write your pallas kernel here")


# speedup vs baseline: 2.2416x; 2.2416x over previous
import functools

import jax
import jax.numpy as jnp
from jax import lax
from jax.experimental import pallas as pl
from jax.experimental.pallas import tpu as pltpu

_NUM_EXPERTS = 8
_TOP_K = 2
_BM = 256


def _matmul_block(be_ref, x_ref, w_ref, b_ref, s_ref, o_ref):
    x = x_ref[...]
    w = w_ref[0]
    y = lax.dot_general(x, w, (((1,), (1,)), ((), ())),
                        preferred_element_type=jnp.float32)
    y = y + b_ref[0]
    o_ref[...] = y * s_ref[...]


def _grouped_matmul(dispatch, W, b, w_col, block_expert, num_blocks, d):
    grid_spec = pltpu.PrefetchScalarGridSpec(
        num_scalar_prefetch=1,
        grid=(num_blocks,),
        in_specs=[
            pl.BlockSpec((_BM, d), lambda i, be: (i, 0)),
            pl.BlockSpec((1, d, d), lambda i, be: (be[i], 0, 0)),
            pl.BlockSpec((1, 1, d), lambda i, be: (be[i], 0, 0)),
            pl.BlockSpec((_BM, 1), lambda i, be: (i, 0)),
        ],
        out_specs=pl.BlockSpec((_BM, d), lambda i, be: (i, 0)),
    )
    return pl.pallas_call(
        _matmul_block,
        grid_spec=grid_spec,
        out_shape=jax.ShapeDtypeStruct((num_blocks * _BM, d), jnp.float32),
    )(block_expert, dispatch, W, b.reshape(b.shape[0], 1, d), w_col)


def kernel(hidden, gate_logits, W, b):
    bsz, seq, d = hidden.shape
    n_tok = bsz * seq
    k = _TOP_K
    e = _NUM_EXPERTS
    n_slots = n_tok * k
    p = n_slots + e * _BM
    num_blocks = p // _BM

    hidden_flat = hidden.reshape(n_tok, d)

    probs = jax.nn.softmax(gate_logits, axis=-1)
    topk_w, topk_i = lax.top_k(probs, k)
    flat_e = topk_i.reshape(-1)

    onehot = (flat_e[:, None] == jnp.arange(e)[None, :]).astype(jnp.int32)
    cum = jnp.cumsum(onehot, axis=0)
    rank = jnp.take_along_axis(cum, flat_e[:, None], axis=1)[:, 0] - 1
    counts = cum[-1]
    padded_counts = ((counts + _BM - 1) // _BM) * _BM
    padded_end = jnp.cumsum(padded_counts)
    padded_start = padded_end - padded_counts
    padded_pos = padded_start[flat_e] + rank

    block_expert = jnp.minimum(
        jnp.sum(jnp.arange(num_blocks)[:, None] * _BM >= padded_end[None, :],
                axis=1), e - 1).astype(jnp.int32)

    tok_of_slot = jnp.arange(n_slots, dtype=jnp.int32) // k
    gather_tok = jnp.zeros((p,), jnp.int32).at[padded_pos].set(tok_of_slot)
    w_col = jnp.zeros((p,), jnp.float32).at[padded_pos].set(
        topk_w.reshape(-1)).reshape(p, 1)

    dispatch = hidden_flat[gather_tok]

    y = _grouped_matmul(dispatch, W, b, w_col, block_expert, num_blocks, d)

    pos = padded_pos.reshape(n_tok, k)
    combined = y[pos[:, 0]] + y[pos[:, 1]]
    return combined.reshape(bsz, seq, d)
